# baseline (speedup 1.0000x reference)
import functools

import jax
import jax.numpy as jnp
from jax import lax
from jax.experimental import pallas as pl
from jax.experimental.pallas import tpu as pltpu
from jax.experimental.pallas import tpu_sc as plsc

_LANES = 16
_CHUNK = 32


def _posenc_sc(embed1d, lut1d, *, rows, d, seq):
    info = plsc.get_sparse_core_info()
    nw = info.num_cores * info.num_subcores
    rows_per_w = rows // nw
    n_chunks = rows_per_w // _CHUNK
    chunk_elems = _CHUNK * d
    mesh = plsc.VectorSubcoreMesh(core_axis_name="c", subcore_axis_name="s")

    @functools.partial(
        pl.kernel,
        mesh=mesh,
        out_type=jax.ShapeDtypeStruct((rows * d,), jnp.float32),
        scratch_types=[
            pltpu.VMEM((chunk_elems,), jnp.float32),
            pltpu.VMEM((chunk_elems,), jnp.float32),
        ],
    )
    def k(embed_hbm, lut_hbm, out_hbm, buf, buf2):
        wid = lax.axis_index("s") * info.num_cores + lax.axis_index("c")
        base = wid * rows_per_w

        def chunk_body(c, carry):
            row0 = base + c * _CHUNK
            e0 = row0 * d
            s0 = lax.rem(row0, seq) * d
            pltpu.sync_copy(embed_hbm.at[pl.ds(e0, chunk_elems)], buf)
            pltpu.sync_copy(lut_hbm.at[pl.ds(s0, chunk_elems)], buf2)

            def row_body(r, carry2):
                off = r * d
                for cc in range(d // _LANES):
                    sl = pl.ds(off + cc * _LANES, _LANES)
                    buf[sl] = buf[sl] + buf2[sl]
                return carry2

            lax.fori_loop(0, _CHUNK, row_body, 0)
            pltpu.sync_copy(buf, out_hbm.at[pl.ds(e0, chunk_elems)])
            return carry

        lax.fori_loop(0, n_chunks, chunk_body, 0)

    return k(embed1d, lut1d)


def kernel(embed, learn_lut):
    B, S, D = embed.shape
    out1d = _posenc_sc(
        embed.reshape(-1), learn_lut[:S].reshape(-1), rows=B * S, d=D, seq=S
    )
    return out1d.reshape(B, S, D)


# speedup vs baseline: 5.6143x; 5.6143x over previous
import jax
import jax.numpy as jnp
from jax.experimental import pallas as pl


def _posenc_add_kernel(e_ref, l_ref, o_ref):
    o_ref[...] = e_ref[...] + l_ref[...]


def kernel(embed, learn_lut):
    B, S, D = embed.shape
    TS = 2048
    TD = 512
    grid = (S // TS, D // TD, B)
    return pl.pallas_call(
        _posenc_add_kernel,
        grid=grid,
        in_specs=[
            pl.BlockSpec((1, TS, TD), lambda i, j, b: (b, i, j)),
            pl.BlockSpec((TS, TD), lambda i, j, b: (i, j)),
        ],
        out_specs=pl.BlockSpec((1, TS, TD), lambda i, j, b: (b, i, j)),
        out_shape=jax.ShapeDtypeStruct((B, S, D), embed.dtype),
    )(embed, learn_lut[:S])


# speedup vs baseline: 6.0546x; 1.0784x over previous
import jax
import jax.numpy as jnp
from jax.experimental import pallas as pl
from jax.experimental.pallas import tpu as pltpu

_CR = 256
_NBUF = 16


def _stream_add_body(e_hbm, l_hbm, o_hbm, in_b, out_b, lut_v, in_sem, out_sem, lut_sem):
    n_rows = e_hbm.shape[0]
    seq = l_hbm.shape[0]
    n_chunks = n_rows // _CR
    lut_chunks = seq // _CR

    pltpu.make_async_copy(l_hbm, lut_v, lut_sem).start()
    for c in range(_NBUF):
        pltpu.make_async_copy(
            e_hbm.at[pl.ds(c * _CR, _CR)], in_b.at[c], in_sem.at[c]
        ).start()
    pltpu.make_async_copy(l_hbm, lut_v, lut_sem).wait()

    for c in range(n_chunks):
        slot = c % _NBUF
        pltpu.make_async_copy(
            e_hbm.at[pl.ds(c * _CR, _CR)], in_b.at[slot], in_sem.at[slot]
        ).wait()
        if c >= _NBUF:
            pltpu.make_async_copy(
                out_b.at[slot],
                o_hbm.at[pl.ds((c - _NBUF) * _CR, _CR)],
                out_sem.at[slot],
            ).wait()
        out_b[slot] = in_b[slot] + lut_v[pl.ds((c % lut_chunks) * _CR, _CR)]
        pltpu.make_async_copy(
            out_b.at[slot], o_hbm.at[pl.ds(c * _CR, _CR)], out_sem.at[slot]
        ).start()
        nxt = c + _NBUF
        if nxt < n_chunks:
            pltpu.make_async_copy(
                e_hbm.at[pl.ds(nxt * _CR, _CR)], in_b.at[slot], in_sem.at[slot]
            ).start()

    for c in range(n_chunks - _NBUF, n_chunks):
        slot = c % _NBUF
        pltpu.make_async_copy(
            out_b.at[slot], o_hbm.at[pl.ds(c * _CR, _CR)], out_sem.at[slot]
        ).wait()


def kernel(embed, learn_lut):
    B, S, D = embed.shape
    out2d = pl.pallas_call(
        _stream_add_body,
        in_specs=[
            pl.BlockSpec(memory_space=pltpu.MemorySpace.HBM),
            pl.BlockSpec(memory_space=pltpu.MemorySpace.HBM),
        ],
        out_specs=pl.BlockSpec(memory_space=pltpu.MemorySpace.HBM),
        out_shape=jax.ShapeDtypeStruct((B * S, D), embed.dtype),
        scratch_shapes=[
            pltpu.VMEM((_NBUF, _CR, D), jnp.float32),
            pltpu.VMEM((_NBUF, _CR, D), jnp.float32),
            pltpu.VMEM((S, D), jnp.float32),
            pltpu.SemaphoreType.DMA((_NBUF,)),
            pltpu.SemaphoreType.DMA((_NBUF,)),
            pltpu.SemaphoreType.DMA,
        ],
    )(embed.reshape(B * S, D), learn_lut[:S])
    return out2d.reshape(B, S, D)


# speedup vs baseline: 6.1738x; 1.0197x over previous
import jax
import jax.numpy as jnp
from jax.experimental import pallas as pl
from jax.experimental.pallas import tpu as pltpu

_CR = 1024
_NBUF = 5


def _stream_add_body(e_hbm, l_hbm, o_hbm, in_b, out_b, lut_v, in_sem, out_sem, lut_sem):
    n_rows = e_hbm.shape[0]
    seq = l_hbm.shape[0]
    n_chunks = n_rows // _CR
    lut_chunks = seq // _CR

    pltpu.make_async_copy(l_hbm, lut_v, lut_sem).start()
    for c in range(_NBUF):
        pltpu.make_async_copy(
            e_hbm.at[pl.ds(c * _CR, _CR)], in_b.at[c], in_sem.at[c]
        ).start()
    pltpu.make_async_copy(l_hbm, lut_v, lut_sem).wait()

    for c in range(n_chunks):
        slot = c % _NBUF
        pltpu.make_async_copy(
            e_hbm.at[pl.ds(c * _CR, _CR)], in_b.at[slot], in_sem.at[slot]
        ).wait()
        if c >= _NBUF:
            pltpu.make_async_copy(
                out_b.at[slot],
                o_hbm.at[pl.ds((c - _NBUF) * _CR, _CR)],
                out_sem.at[slot],
            ).wait()
        out_b[slot] = in_b[slot] + lut_v[pl.ds((c % lut_chunks) * _CR, _CR)]
        pltpu.make_async_copy(
            out_b.at[slot], o_hbm.at[pl.ds(c * _CR, _CR)], out_sem.at[slot]
        ).start()
        nxt = c + _NBUF
        if nxt < n_chunks:
            pltpu.make_async_copy(
                e_hbm.at[pl.ds(nxt * _CR, _CR)], in_b.at[slot], in_sem.at[slot]
            ).start()

    for c in range(n_chunks - _NBUF, n_chunks):
        slot = c % _NBUF
        pltpu.make_async_copy(
            out_b.at[slot], o_hbm.at[pl.ds(c * _CR, _CR)], out_sem.at[slot]
        ).wait()


def kernel(embed, learn_lut):
    B, S, D = embed.shape
    out2d = pl.pallas_call(
        _stream_add_body,
        in_specs=[
            pl.BlockSpec(memory_space=pltpu.MemorySpace.HBM),
            pl.BlockSpec(memory_space=pltpu.MemorySpace.HBM),
        ],
        out_specs=pl.BlockSpec(memory_space=pltpu.MemorySpace.HBM),
        out_shape=jax.ShapeDtypeStruct((B * S, D), embed.dtype),
        scratch_shapes=[
            pltpu.VMEM((_NBUF, _CR, D), jnp.float32),
            pltpu.VMEM((_NBUF, _CR, D), jnp.float32),
            pltpu.VMEM((S, D), jnp.float32),
            pltpu.SemaphoreType.DMA((_NBUF,)),
            pltpu.SemaphoreType.DMA((_NBUF,)),
            pltpu.SemaphoreType.DMA,
        ],
    )(embed.reshape(B * S, D), learn_lut[:S])
    return out2d.reshape(B, S, D)
